# baseline (speedup 1.0000x reference)
import functools

import jax
import jax.numpy as jnp
from jax import lax
from jax.experimental import pallas as pl
from jax.experimental.pallas import tpu as pltpu
from jax.experimental.pallas import tpu_sc as plsc

D_MODEL_ = 1024
NUM_EXPERTS_ = 8
D_FF_ = 2048
N_TOK_ = 32
LANES_ = 16
NEG_INF_ = float("-inf")


def _logits_kernel(x_ref, rw_ref, out_ref):
    out_ref[...] = jax.lax.dot_general(
        rw_ref[...], x_ref[...], (((1,), (1,)), ((), ())),
        preferred_element_type=jnp.float32)


def _routing_sc_kernel(lt_hbm, c_hbm, lt_v, c_v):
    cid = lax.axis_index("c")
    sid = lax.axis_index("s")

    @pl.when((sid == 0) & (cid == 0))
    def _():
        pltpu.sync_copy(lt_hbm, lt_v)
        for half in range(N_TOK_ // LANES_):
            m1 = jnp.full((LANES_,), NEG_INF_, jnp.float32)
            m2 = jnp.full((LANES_,), NEG_INF_, jnp.float32)
            i1 = jnp.zeros((LANES_,), jnp.int32)
            i2 = jnp.zeros((LANES_,), jnp.int32)
            for e in range(NUM_EXPERTS_):
                v = lt_v[pl.ds(e * N_TOK_ + half * LANES_, LANES_)]
                ev = jnp.full((LANES_,), e, jnp.int32)
                beats1 = v > m1
                beats2 = v > m2
                i2 = jnp.where(beats1, i1, jnp.where(beats2, ev, i2))
                m2 = jnp.where(beats1, m1, jnp.where(beats2, v, m2))
                i1 = jnp.where(beats1, ev, i1)
                m1 = jnp.where(beats1, v, m1)
            p1 = 1.0 / (1.0 + jnp.exp(m2 - m1))
            p2 = 1.0 - p1
            for e in range(NUM_EXPERTS_):
                ev = jnp.full((LANES_,), e, jnp.int32)
                c_v[pl.ds(e * N_TOK_ + half * LANES_, LANES_)] = (
                    jnp.where(i1 == ev, p1, 0.0)
                    + jnp.where(i2 == ev, p2, 0.0))
        pltpu.sync_copy(c_v, c_hbm)


def _routing_sc(logits_t_flat):
    mesh = plsc.VectorSubcoreMesh(core_axis_name="c", subcore_axis_name="s")
    f = functools.partial(
        pl.kernel,
        mesh=mesh,
        out_type=jax.ShapeDtypeStruct((NUM_EXPERTS_ * N_TOK_,), jnp.float32),
        scratch_types=[
            pltpu.VMEM((NUM_EXPERTS_ * N_TOK_,), jnp.float32),
            pltpu.VMEM((NUM_EXPERTS_ * N_TOK_,), jnp.float32),
        ],
    )(_routing_sc_kernel)
    return f(logits_t_flat)


def _ffn_kernel(x_ref, w1_ref, w2_ref, out_ref):
    x = x_ref[...]
    w1_e = w1_ref[0]
    w2_e = w2_ref[0]
    h = jax.lax.dot_general(x, w1_e, (((1,), (1,)), ((), ())),
                            preferred_element_type=jnp.float32)
    h = 0.5 * h * (1.0 + jax.lax.erf(h * (2.0 ** -0.5)))
    y = jax.lax.dot_general(h, w2_e, (((1,), (1,)), ((), ())),
                            preferred_element_type=jnp.float32)
    out_ref[0] = y


def _combine_kernel(parts_ref, c_ref, out_ref):
    out_ref[...] = jnp.sum(c_ref[...][:, :, None] * parts_ref[...], axis=0)


@jax.jit
def _moe(x_flat, w1, w2, router_w):
    n = x_flat.shape[0]
    logits_t = pl.pallas_call(
        _logits_kernel,
        out_shape=jax.ShapeDtypeStruct((NUM_EXPERTS_, n), jnp.float32),
    )(x_flat, router_w)
    c = _routing_sc(logits_t.reshape(-1)).reshape(NUM_EXPERTS_, n)
    parts = pl.pallas_call(
        _ffn_kernel,
        grid=(NUM_EXPERTS_,),
        in_specs=[
            pl.BlockSpec((n, D_MODEL_), lambda e: (0, 0)),
            pl.BlockSpec((1, D_FF_, D_MODEL_), lambda e: (e, 0, 0)),
            pl.BlockSpec((1, D_MODEL_, D_FF_), lambda e: (e, 0, 0)),
        ],
        out_specs=pl.BlockSpec((1, n, D_MODEL_), lambda e: (e, 0, 0)),
        out_shape=jax.ShapeDtypeStruct((NUM_EXPERTS_, n, D_MODEL_),
                                       jnp.float32),
    )(x_flat, w1, w2)
    return pl.pallas_call(
        _combine_kernel,
        out_shape=jax.ShapeDtypeStruct((n, D_MODEL_), jnp.float32),
    )(parts, c)


def kernel(x, w1, w2, router_w):
    B, T, D = x.shape
    out = _moe(x.reshape(B * T, D), w1, w2, router_w)
    return out.reshape(B, T, D)


# speedup vs baseline: 1.4283x; 1.4283x over previous
import jax
import jax.numpy as jnp
from jax.experimental import pallas as pl
from jax.experimental.pallas import tpu as pltpu

D_MODEL_ = 1024
NUM_EXPERTS_ = 8
D_FF_ = 2048


def _moe_kernel(x_ref, w1_ref, w2_ref, rw_ref, out_ref, c_scr):
    e = pl.program_id(0)
    x = x_ref[...]
    n = x.shape[0]
    col = jax.lax.broadcasted_iota(jnp.int32, (n, NUM_EXPERTS_), 1)

    @pl.when(e == 0)
    def _():
        logits = jax.lax.dot_general(
            x, rw_ref[...], (((1,), (1,)), ((), ())),
            preferred_element_type=jnp.float32)
        m1 = jnp.max(logits, axis=1, keepdims=True)
        i1 = jnp.min(jnp.where(logits == m1, col, NUM_EXPERTS_), axis=1,
                     keepdims=True)
        masked = jnp.where(col == i1, -jnp.inf, logits)
        m2 = jnp.max(masked, axis=1, keepdims=True)
        i2 = jnp.min(jnp.where(masked == m2, col, NUM_EXPERTS_), axis=1,
                     keepdims=True)
        p1 = 1.0 / (1.0 + jnp.exp(m2 - m1))
        p2 = 1.0 - p1
        c_scr[...] = (jnp.where(col == i1, p1, 0.0)
                      + jnp.where(col == i2, p2, 0.0))

    c_e = jnp.sum(jnp.where(col == e, c_scr[...], 0.0), axis=1)

    w1_e = w1_ref[0]
    w2_e = w2_ref[0]
    h = jax.lax.dot_general(x, w1_e, (((1,), (1,)), ((), ())),
                            preferred_element_type=jnp.float32)
    h = 0.5 * h * (1.0 + jax.lax.erf(h * (2.0 ** -0.5)))
    y = jax.lax.dot_general(h, w2_e, (((1,), (1,)), ((), ())),
                            preferred_element_type=jnp.float32)

    contrib = c_e[:, None] * y

    @pl.when(e == 0)
    def _():
        out_ref[...] = contrib

    @pl.when(e > 0)
    def _():
        out_ref[...] += contrib


@jax.jit
def _moe(x_flat, w1, w2, router_w):
    n = x_flat.shape[0]
    return pl.pallas_call(
        _moe_kernel,
        grid=(NUM_EXPERTS_,),
        in_specs=[
            pl.BlockSpec((n, D_MODEL_), lambda e: (0, 0)),
            pl.BlockSpec((1, D_FF_, D_MODEL_), lambda e: (e, 0, 0)),
            pl.BlockSpec((1, D_MODEL_, D_FF_), lambda e: (e, 0, 0)),
            pl.BlockSpec((NUM_EXPERTS_, D_MODEL_), lambda e: (0, 0)),
        ],
        out_specs=pl.BlockSpec((n, D_MODEL_), lambda e: (0, 0)),
        out_shape=jax.ShapeDtypeStruct((n, D_MODEL_), jnp.float32),
        scratch_shapes=[pltpu.VMEM((n, NUM_EXPERTS_), jnp.float32)],
    )(x_flat, w1, w2, router_w)


def kernel(x, w1, w2, router_w):
    B, T, D = x.shape
    out = _moe(x.reshape(B * T, D), w1, w2, router_w)
    return out.reshape(B, T, D)


# speedup vs baseline: 1.4291x; 1.0006x over previous
import jax
import jax.numpy as jnp
from jax.experimental import pallas as pl

D_MODEL_ = 1024
NUM_EXPERTS_ = 8
D_FF_ = 2048


def _moe_kernel(x_ref, w1_ref, w2_ref, rw_ref, out_ref):
    e = pl.program_id(0)
    x = x_ref[...]

    logits = jax.lax.dot_general(
        x, rw_ref[...], (((1,), (1,)), ((), ())),
        preferred_element_type=jnp.float32)
    col = jax.lax.broadcasted_iota(jnp.int32, logits.shape, 1)
    m1 = jnp.max(logits, axis=1, keepdims=True)
    i1 = jnp.min(jnp.where(logits == m1, col, NUM_EXPERTS_), axis=1,
                 keepdims=True)
    masked = jnp.where(col == i1, -jnp.inf, logits)
    m2 = jnp.max(masked, axis=1, keepdims=True)
    i2 = jnp.min(jnp.where(masked == m2, col, NUM_EXPERTS_), axis=1,
                 keepdims=True)
    p1 = 1.0 / (1.0 + jnp.exp(m2 - m1))
    p2 = 1.0 - p1
    c_e = jnp.sum(jnp.where(col == i1, p1, 0.0) * (col == e)
                  + jnp.where(col == i2, p2, 0.0) * (col == e), axis=1)

    w1_e = w1_ref[0]
    w2_e = w2_ref[0]
    h = jax.lax.dot_general(x, w1_e, (((1,), (1,)), ((), ())),
                            preferred_element_type=jnp.float32)
    h = 0.5 * h * (1.0 + jax.lax.erf(h * (2.0 ** -0.5)))
    y = jax.lax.dot_general(h, w2_e, (((1,), (1,)), ((), ())),
                            preferred_element_type=jnp.float32)

    contrib = c_e[:, None] * y

    @pl.when(e == 0)
    def _():
        out_ref[...] = contrib

    @pl.when(e > 0)
    def _():
        out_ref[...] += contrib


@jax.jit
def _moe(x_flat, w1, w2, router_w):
    n = x_flat.shape[0]
    return pl.pallas_call(
        _moe_kernel,
        grid=(NUM_EXPERTS_,),
        in_specs=[
            pl.BlockSpec((n, D_MODEL_), lambda e: (0, 0)),
            pl.BlockSpec((1, D_FF_, D_MODEL_), lambda e: (e, 0, 0)),
            pl.BlockSpec((1, D_MODEL_, D_FF_), lambda e: (e, 0, 0)),
            pl.BlockSpec((NUM_EXPERTS_, D_MODEL_), lambda e: (0, 0)),
        ],
        out_specs=pl.BlockSpec((n, D_MODEL_), lambda e: (0, 0)),
        out_shape=jax.ShapeDtypeStruct((n, D_MODEL_), jnp.float32),
    )(x_flat, w1, w2, router_w)


def kernel(x, w1, w2, router_w):
    B, T, D = x.shape
    out = _moe(x.reshape(B * T, D), w1, w2, router_w)
    return out.reshape(B, T, D)


# speedup vs baseline: 1.5140x; 1.0594x over previous
import jax
import jax.numpy as jnp
from jax.experimental import pallas as pl
from jax.experimental.pallas import tpu as pltpu

D_MODEL_ = 1024
NUM_EXPERTS_ = 8
D_FF_ = 2048


def _moe_kernel(x_ref, rw_ref, w1_hbm, w2_hbm, out_ref,
                w1_buf, w2_buf, s1, s2):
    e = pl.program_id(0)
    x = x_ref[...]

    @pl.when(e == 0)
    def _():
        pltpu.make_async_copy(w1_hbm.at[0], w1_buf.at[0], s1.at[0]).start()
        pltpu.make_async_copy(w2_hbm.at[0], w2_buf.at[0], s2.at[0]).start()

    @pl.when(e < NUM_EXPERTS_ - 1)
    def _():
        nxt = (e + 1) % 2

        @pl.when(nxt == 1)
        def _():
            pltpu.make_async_copy(w1_hbm.at[e + 1], w1_buf.at[1],
                                  s1.at[1]).start()
            pltpu.make_async_copy(w2_hbm.at[e + 1], w2_buf.at[1],
                                  s2.at[1]).start()

        @pl.when(nxt == 0)
        def _():
            pltpu.make_async_copy(w1_hbm.at[e + 1], w1_buf.at[0],
                                  s1.at[0]).start()
            pltpu.make_async_copy(w2_hbm.at[e + 1], w2_buf.at[0],
                                  s2.at[0]).start()

    logits = jax.lax.dot_general(
        x, rw_ref[...], (((1,), (1,)), ((), ())),
        preferred_element_type=jnp.float32)
    col = jax.lax.broadcasted_iota(jnp.int32, logits.shape, 1)
    m1 = jnp.max(logits, axis=1, keepdims=True)
    i1 = jnp.min(jnp.where(logits == m1, col, NUM_EXPERTS_), axis=1,
                 keepdims=True)
    masked = jnp.where(col == i1, -jnp.inf, logits)
    m2 = jnp.max(masked, axis=1, keepdims=True)
    i2 = jnp.min(jnp.where(masked == m2, col, NUM_EXPERTS_), axis=1,
                 keepdims=True)
    p1 = 1.0 / (1.0 + jnp.exp(m2 - m1))
    p2 = 1.0 - p1
    c_e = jnp.sum(jnp.where(col == i1, p1, 0.0) * (col == e)
                  + jnp.where(col == i2, p2, 0.0) * (col == e), axis=1)

    def ffn(slot):
        pltpu.make_async_copy(w1_hbm.at[e], w1_buf.at[slot],
                              s1.at[slot]).wait()
        h = jax.lax.dot_general(x, w1_buf[slot], (((1,), (1,)), ((), ())),
                                preferred_element_type=jnp.float32)
        h = 0.5 * h * (1.0 + jax.lax.erf(h * (2.0 ** -0.5)))
        pltpu.make_async_copy(w2_hbm.at[e], w2_buf.at[slot],
                              s2.at[slot]).wait()
        y = jax.lax.dot_general(h, w2_buf[slot], (((1,), (1,)), ((), ())),
                                preferred_element_type=jnp.float32)
        contrib = c_e[:, None] * y

        @pl.when(e == 0)
        def _():
            out_ref[...] = contrib

        @pl.when(e > 0)
        def _():
            out_ref[...] += contrib

    @pl.when(e % 2 == 0)
    def _():
        ffn(0)

    @pl.when(e % 2 == 1)
    def _():
        ffn(1)


@jax.jit
def _moe(x_flat, w1, w2, router_w):
    n = x_flat.shape[0]
    return pl.pallas_call(
        _moe_kernel,
        grid=(NUM_EXPERTS_,),
        in_specs=[
            pl.BlockSpec((n, D_MODEL_), lambda e: (0, 0)),
            pl.BlockSpec((NUM_EXPERTS_, D_MODEL_), lambda e: (0, 0)),
            pl.BlockSpec(memory_space=pl.ANY),
            pl.BlockSpec(memory_space=pl.ANY),
        ],
        out_specs=pl.BlockSpec((n, D_MODEL_), lambda e: (0, 0)),
        out_shape=jax.ShapeDtypeStruct((n, D_MODEL_), jnp.float32),
        scratch_shapes=[
            pltpu.VMEM((2, D_FF_, D_MODEL_), jnp.float32),
            pltpu.VMEM((2, D_MODEL_, D_FF_), jnp.float32),
            pltpu.SemaphoreType.DMA((2,)),
            pltpu.SemaphoreType.DMA((2,)),
        ],
    )(x_flat, router_w, w1, w2)


def kernel(x, w1, w2, router_w):
    B, T, D = x.shape
    out = _moe(x.reshape(B * T, D), w1, w2, router_w)
    return out.reshape(B, T, D)
